# baseline (speedup 1.0000x reference)
import functools

import jax
import jax.numpy as jnp
import numpy as np
from jax import lax
from jax.experimental import pallas as pl
from jax.experimental.pallas import tpu as pltpu
from jax.experimental.pallas import tpu_sc as plsc

T, B, C = 576, 128, 768
R = T * B
NC, NS = 2, 16
NW = NC * NS
RPW = R // NW
NBUF = 3
CHUNK = 48
NCHUNK = RPW // CHUNK
NROUND = NCHUNK // NBUF


def _rotl32(x, d):
    return (x << np.uint32(d)) | (x >> np.uint32(32 - d))


def _threefry2x32(k1, k2, x0, x1):
    rot = ((13, 15, 26, 6), (17, 29, 16, 24))
    ks = (np.uint32(k1), np.uint32(k2),
          np.uint32(k1) ^ np.uint32(k2) ^ np.uint32(0x1BD11BDA))
    x0 = (x0 + ks[0]).astype(np.uint32)
    x1 = (x1 + ks[1]).astype(np.uint32)
    inj = ((1, 2, 1), (2, 0, 2), (0, 1, 3), (1, 2, 4), (2, 0, 5))
    for grp in range(5):
        for r in rot[grp % 2]:
            x0 = (x0 + x1).astype(np.uint32)
            x1 = x0 ^ _rotl32(x1, r)
        a, b, c = inj[grp]
        x0 = (x0 + ks[a]).astype(np.uint32)
        x1 = (x1 + ks[b] + np.uint32(c)).astype(np.uint32)
    return x0, x1


def _iota_counts(n):
    cnt = np.arange(n, dtype=np.uint64)
    return ((cnt >> np.uint64(32)).astype(np.uint32),
            (cnt & np.uint64(0xFFFFFFFF)).astype(np.uint32))


def _np_split(k1, k2, num, partitionable):
    if partitionable:
        b1, b2 = _threefry2x32(k1, k2, *_iota_counts(num))
        return np.stack([b1, b2], axis=1)
    counts = np.arange(num * 2, dtype=np.uint32)
    o0, o1 = _threefry2x32(k1, k2, counts[:num], counts[num:])
    return np.concatenate([o0, o1]).reshape(num, 2)


def _np_bits32(k1, k2, n, partitionable):
    if partitionable:
        b1, b2 = _threefry2x32(k1, k2, *_iota_counts(n))
        return b1 ^ b2
    h = (n + 1) // 2
    counts = np.arange(2 * h, dtype=np.uint32)
    counts[n:] = 0
    o0, o1 = _threefry2x32(k1, k2, counts[:h], counts[h:])
    return np.concatenate([o0, o1])[:n]


def _make_indexes():
    part = bool(jax.config.jax_threefry_partitionable)
    seed = 42
    k1 = np.uint32(np.uint64(seed) >> np.uint64(32))
    k2 = np.uint32(seed & 0xFFFFFFFF)
    keys = _np_split(k1, k2, B, part)
    cols = []
    for i in range(B):
        sub = _np_split(keys[i, 0], keys[i, 1], 2, part)[1]
        bits = _np_bits32(sub[0], sub[1], T, part)
        cols.append(np.argsort(bits, kind="stable").astype(np.int32))
    fwd = np.stack(cols, axis=-1)
    bwd = np.argsort(fwd, axis=0, kind="stable").astype(np.int32)
    flat = np.arange(R, dtype=np.int32)
    return fwd, bwd, flat


_FWD, _BWD, _IDX_FLAT = _make_indexes()

@functools.cache
def _build_permute_rows():
    mesh = plsc.VectorSubcoreMesh(core_axis_name="c", subcore_axis_name="s",
                                  num_cores=NC, num_subcores=NS)
    return functools.partial(
        pl.kernel,
        out_type=jax.ShapeDtypeStruct((R, C), jnp.float32),
        mesh=mesh,
        scratch_types=(
            [pltpu.VMEM((RPW,), jnp.int32)]
            + [pltpu.VMEM((CHUNK, C), jnp.float32)] * NBUF
            + [pltpu.SemaphoreType.DMA] * NBUF
            + [pltpu.SemaphoreType.DMA] * NBUF
        ),
    )(_permute_rows_body)


def _permute_rows_body(src, idxh, out, idx_v, *rest):
    bufs_ = rest[:NBUF]
    isems = rest[NBUF:2 * NBUF]
    osems = rest[2 * NBUF:3 * NBUF]
    wid = lax.axis_index("s") * NC + lax.axis_index("c")
    base = pl.multiple_of(wid * RPW, RPW)
    pltpu.sync_copy(idxh.at[pl.ds(base, RPW)], idx_v)

    def g_src(c):
        return src.at[idx_v.at[pl.ds(pl.multiple_of(c * CHUNK, 8), CHUNK)]]

    def o_dst(c):
        return out.at[pl.ds(pl.multiple_of(base + c * CHUNK, 8), CHUNK)]

    bufs = tuple(zip(bufs_, isems, osems))

    for b, (buf, isem, _) in enumerate(bufs):
        pltpu.async_copy(g_src(b), buf, isem)

    def step(i, carry):
        for b, (buf, isem, osem) in enumerate(bufs):
            c = NBUF * i + b
            pltpu.make_async_copy(g_src(c), buf, isem).wait()
            pltpu.async_copy(buf, o_dst(c), osem)

            @pl.when(c + NBUF < NCHUNK)
            def _():
                pltpu.make_async_copy(buf, o_dst(c), osem).wait()
                pltpu.async_copy(g_src(c + NBUF), buf, isem)
        return carry

    lax.fori_loop(0, NROUND, step, 0)
    for b, (buf, _, osem) in enumerate(bufs):
        pltpu.make_async_copy(buf, o_dst(NCHUNK - NBUF + b), osem).wait()


def kernel(patches):
    src = patches.reshape(R, C)
    shuffled = _build_permute_rows()(src, jnp.asarray(_IDX_FLAT)).reshape(T, B, C)
    fwd = jnp.asarray(_FWD).astype(jnp.int64)
    bwd = jnp.asarray(_BWD).astype(jnp.int64)
    return shuffled, fwd, bwd


# speedup vs baseline: 1.0075x; 1.0075x over previous
import functools

import jax
import jax.numpy as jnp
import numpy as np
from jax import lax
from jax.experimental import pallas as pl
from jax.experimental.pallas import tpu as pltpu
from jax.experimental.pallas import tpu_sc as plsc

T, B, C = 576, 128, 768
R = T * B
NC, NS = 2, 16
NW = NC * NS
RPW = R // NW
NBUF = 3
CHUNK = 48
NCHUNK = RPW // CHUNK
NROUND = NCHUNK // NBUF


def _rotl32(x, d):
    return (x << np.uint32(d)) | (x >> np.uint32(32 - d))


def _threefry2x32(k1, k2, x0, x1):
    rot = ((13, 15, 26, 6), (17, 29, 16, 24))
    ks = (np.uint32(k1), np.uint32(k2),
          np.uint32(k1) ^ np.uint32(k2) ^ np.uint32(0x1BD11BDA))
    x0 = (x0 + ks[0]).astype(np.uint32)
    x1 = (x1 + ks[1]).astype(np.uint32)
    inj = ((1, 2, 1), (2, 0, 2), (0, 1, 3), (1, 2, 4), (2, 0, 5))
    for grp in range(5):
        for r in rot[grp % 2]:
            x0 = (x0 + x1).astype(np.uint32)
            x1 = x0 ^ _rotl32(x1, r)
        a, b, c = inj[grp]
        x0 = (x0 + ks[a]).astype(np.uint32)
        x1 = (x1 + ks[b] + np.uint32(c)).astype(np.uint32)
    return x0, x1


def _iota_counts(n):
    cnt = np.arange(n, dtype=np.uint64)
    return ((cnt >> np.uint64(32)).astype(np.uint32),
            (cnt & np.uint64(0xFFFFFFFF)).astype(np.uint32))


def _np_split(k1, k2, num, partitionable):
    if partitionable:
        b1, b2 = _threefry2x32(k1, k2, *_iota_counts(num))
        return np.stack([b1, b2], axis=1)
    counts = np.arange(num * 2, dtype=np.uint32)
    o0, o1 = _threefry2x32(k1, k2, counts[:num], counts[num:])
    return np.concatenate([o0, o1]).reshape(num, 2)


def _np_bits32(k1, k2, n, partitionable):
    if partitionable:
        b1, b2 = _threefry2x32(k1, k2, *_iota_counts(n))
        return b1 ^ b2
    h = (n + 1) // 2
    counts = np.arange(2 * h, dtype=np.uint32)
    counts[n:] = 0
    o0, o1 = _threefry2x32(k1, k2, counts[:h], counts[h:])
    return np.concatenate([o0, o1])[:n]


def _make_indexes():
    part = bool(jax.config.jax_threefry_partitionable)
    seed = 42
    k1 = np.uint32(np.uint64(seed) >> np.uint64(32))
    k2 = np.uint32(seed & 0xFFFFFFFF)
    keys = _np_split(k1, k2, B, part)
    cols = []
    for i in range(B):
        sub = _np_split(keys[i, 0], keys[i, 1], 2, part)[1]
        bits = _np_bits32(sub[0], sub[1], T, part)
        cols.append(np.argsort(bits, kind="stable").astype(np.int32))
    fwd = np.stack(cols, axis=-1)
    bwd = np.argsort(fwd, axis=0, kind="stable").astype(np.int32)
    flat = (fwd * B + np.arange(B, dtype=np.int32)[None, :]).reshape(R)
    return fwd, bwd, flat


_FWD, _BWD, _IDX_FLAT = _make_indexes()

@functools.cache
def _build_permute_rows():
    mesh = plsc.VectorSubcoreMesh(core_axis_name="c", subcore_axis_name="s",
                                  num_cores=NC, num_subcores=NS)
    return functools.partial(
        pl.kernel,
        out_type=jax.ShapeDtypeStruct((R, C), jnp.float32),
        mesh=mesh,
        scratch_types=(
            [pltpu.VMEM((RPW,), jnp.int32)]
            + [pltpu.VMEM((CHUNK, C), jnp.float32)] * NBUF
            + [pltpu.SemaphoreType.DMA] * NBUF
            + [pltpu.SemaphoreType.DMA] * NBUF
        ),
    )(_permute_rows_body)


def _permute_rows_body(src, idxh, out, idx_v, *rest):
    bufs_ = rest[:NBUF]
    isems = rest[NBUF:2 * NBUF]
    osems = rest[2 * NBUF:3 * NBUF]
    wid = lax.axis_index("s") * NC + lax.axis_index("c")
    base = pl.multiple_of(wid * RPW, RPW)
    pltpu.sync_copy(idxh.at[pl.ds(base, RPW)], idx_v)

    def g_src(c):
        return src.at[idx_v.at[pl.ds(pl.multiple_of(c * CHUNK, 8), CHUNK)]]

    def o_dst(c):
        return out.at[pl.ds(pl.multiple_of(base + c * CHUNK, 8), CHUNK)]

    bufs = tuple(zip(bufs_, isems, osems))

    for b, (buf, isem, _) in enumerate(bufs):
        pltpu.async_copy(g_src(b), buf, isem)

    def step(i, carry):
        for b, (buf, isem, osem) in enumerate(bufs):
            c = NBUF * i + b
            pltpu.make_async_copy(g_src(c), buf, isem).wait()
            pltpu.async_copy(buf, o_dst(c), osem)

            @pl.when(c + NBUF < NCHUNK)
            def _():
                pltpu.make_async_copy(buf, o_dst(c), osem).wait()
                pltpu.async_copy(g_src(c + NBUF), buf, isem)
        return carry

    lax.fori_loop(0, NROUND, step, 0)
    for b, (buf, _, osem) in enumerate(bufs):
        pltpu.make_async_copy(buf, o_dst(NCHUNK - NBUF + b), osem).wait()


def kernel(patches):
    src = patches.reshape(R, C)
    shuffled = _build_permute_rows()(src, jnp.asarray(_IDX_FLAT)).reshape(T, B, C)
    fwd = jnp.asarray(_FWD).astype(jnp.int64)
    bwd = jnp.asarray(_BWD).astype(jnp.int64)
    return shuffled, fwd, bwd
